# baseline (speedup 1.0000x reference)
import jax
import jax.numpy as jnp
from jax.experimental import pallas as pl
from jax.experimental.pallas import tpu as pltpu

N = 4096
M = 4096
RB = 128
NB = 8
NCH = M // RB


def _body(a_hbm, x_hbm, b_hbm, iy_hbm, out_ref,
          abuf, x_v, b_v, iy_v, *sems):
    pltpu.make_async_copy(x_hbm, x_v, sems[NB]).start()
    pltpu.make_async_copy(b_hbm, b_v, sems[NB + 1]).start()
    pltpu.make_async_copy(iy_hbm, iy_v, sems[NB + 2]).start()

    def chunk_copy(g, bi):
        return pltpu.make_async_copy(
            a_hbm.at[pl.ds(g * RB, RB)], abuf.at[bi], sems[bi])

    for bi in range(NB):
        chunk_copy(bi, bi).start()

    pltpu.make_async_copy(x_hbm, x_v, sems[NB]).wait()
    pltpu.make_async_copy(b_hbm, b_v, sems[NB + 1]).wait()
    pltpu.make_async_copy(iy_hbm, iy_v, sems[NB + 2]).wait()
    xr = x_v[...]

    @pl.loop(0, NCH, init_carry=jnp.float32(0.0), step=NB)
    def tot(g, tot):
        for bi in range(NB):
            gg = g + bi
            chunk_copy(gg, bi).wait()
            ax = jnp.sum(abuf[bi] * xr, axis=1)
            v = ax - b_v[0, pl.ds(gg * RB, RB)]
            f = v + iy_v[0, pl.ds(gg * RB, RB)] * jnp.maximum(-v, 0.0)
            tot = tot + jnp.sum(jnp.abs(f))

            @pl.when(gg + NB < NCH)
            def _(gg=gg, bi=bi):
                chunk_copy(gg + NB, bi).start()
        return tot

    bsum = jnp.sum(jnp.abs(b_v[...]))
    out_ref[...] = jnp.full((1, 1), tot / (1.0 + bsum), jnp.float32)


_call = pl.pallas_call(
    _body,
    in_specs=[
        pl.BlockSpec(memory_space=pl.ANY),
        pl.BlockSpec(memory_space=pl.ANY),
        pl.BlockSpec(memory_space=pl.ANY),
        pl.BlockSpec(memory_space=pl.ANY),
    ],
    out_specs=pl.BlockSpec(memory_space=pltpu.MemorySpace.VMEM),
    out_shape=jax.ShapeDtypeStruct((1, 1), jnp.float32),
    scratch_shapes=[
        pltpu.VMEM((NB, RB, N), jnp.float32),
        pltpu.VMEM((1, N), jnp.float32),
        pltpu.VMEM((1, M), jnp.float32),
        pltpu.VMEM((1, M), jnp.float32),
    ] + [pltpu.SemaphoreType.DMA] * (NB + 3),
)


def kernel(Q, A, AT, b, c, x, y, Iy):
    res = _call(A, x.reshape(1, N), b.reshape(1, M), Iy.reshape(1, M))
    return res[0, 0]


# speedup vs baseline: 1.1187x; 1.1187x over previous
import jax
import jax.numpy as jnp
from jax.experimental import pallas as pl
from jax.experimental.pallas import tpu as pltpu

N = 4096
M = 4096
RB = 256
NB = 4
NCH = M // RB


def _body(a_hbm, x_hbm, b_hbm, iy_hbm, out_ref,
          abuf, x_v, b_v, iy_v, *sems):
    pltpu.make_async_copy(x_hbm, x_v, sems[NB]).start()
    pltpu.make_async_copy(b_hbm, b_v, sems[NB + 1]).start()
    pltpu.make_async_copy(iy_hbm, iy_v, sems[NB + 2]).start()

    def chunk_copy(g, bi):
        return pltpu.make_async_copy(
            a_hbm.at[pl.ds(g * RB, RB)], abuf.at[bi], sems[bi])

    for bi in range(NB):
        chunk_copy(bi, bi).start()

    pltpu.make_async_copy(x_hbm, x_v, sems[NB]).wait()
    pltpu.make_async_copy(b_hbm, b_v, sems[NB + 1]).wait()
    pltpu.make_async_copy(iy_hbm, iy_v, sems[NB + 2]).wait()
    xr = x_v[...]

    @pl.loop(0, NCH, init_carry=jnp.float32(0.0), step=NB)
    def tot(g, tot):
        for bi in range(NB):
            gg = g + bi
            chunk_copy(gg, bi).wait()
            ax = jnp.sum(abuf[bi] * xr, axis=1)
            v = ax - b_v[0, pl.ds(gg * RB, RB)]
            f = v + iy_v[0, pl.ds(gg * RB, RB)] * jnp.maximum(-v, 0.0)
            tot = tot + jnp.sum(jnp.abs(f))

            @pl.when(gg + NB < NCH)
            def _(gg=gg, bi=bi):
                chunk_copy(gg + NB, bi).start()
        return tot

    bsum = jnp.sum(jnp.abs(b_v[...]))
    out_ref[...] = jnp.full((1, 1), tot / (1.0 + bsum), jnp.float32)


_call = pl.pallas_call(
    _body,
    in_specs=[
        pl.BlockSpec(memory_space=pl.ANY),
        pl.BlockSpec(memory_space=pl.ANY),
        pl.BlockSpec(memory_space=pl.ANY),
        pl.BlockSpec(memory_space=pl.ANY),
    ],
    out_specs=pl.BlockSpec(memory_space=pltpu.MemorySpace.VMEM),
    out_shape=jax.ShapeDtypeStruct((1, 1), jnp.float32),
    scratch_shapes=[
        pltpu.VMEM((NB, RB, N), jnp.float32),
        pltpu.VMEM((1, N), jnp.float32),
        pltpu.VMEM((1, M), jnp.float32),
        pltpu.VMEM((1, M), jnp.float32),
    ] + [pltpu.SemaphoreType.DMA] * (NB + 3),
)


def kernel(Q, A, AT, b, c, x, y, Iy):
    res = _call(A, x.reshape(1, N), b.reshape(1, M), Iy.reshape(1, M))
    return res[0, 0]


# speedup vs baseline: 1.1390x; 1.0181x over previous
import jax
import jax.numpy as jnp
from jax.experimental import pallas as pl
from jax.experimental.pallas import tpu as pltpu

N = 4096
M = 4096
RB = 512
NB = 4
NCH = M // RB


def _body(a_hbm, x_hbm, b_hbm, iy_hbm, out_ref,
          abuf, x_v, b_v, iy_v, *sems):
    pltpu.make_async_copy(x_hbm, x_v, sems[NB]).start()
    pltpu.make_async_copy(b_hbm, b_v, sems[NB + 1]).start()
    pltpu.make_async_copy(iy_hbm, iy_v, sems[NB + 2]).start()

    def chunk_copy(g, bi):
        return pltpu.make_async_copy(
            a_hbm.at[pl.ds(g * RB, RB)], abuf.at[bi], sems[bi])

    for bi in range(NB):
        chunk_copy(bi, bi).start()

    pltpu.make_async_copy(x_hbm, x_v, sems[NB]).wait()
    pltpu.make_async_copy(b_hbm, b_v, sems[NB + 1]).wait()
    pltpu.make_async_copy(iy_hbm, iy_v, sems[NB + 2]).wait()
    xr = x_v[...]

    @pl.loop(0, NCH, init_carry=jnp.float32(0.0), step=NB)
    def tot(g, tot):
        for bi in range(NB):
            gg = g + bi
            chunk_copy(gg, bi).wait()
            ax = jnp.sum(abuf[bi] * xr, axis=1)
            v = ax - b_v[0, pl.ds(gg * RB, RB)]
            f = v + iy_v[0, pl.ds(gg * RB, RB)] * jnp.maximum(-v, 0.0)
            tot = tot + jnp.sum(jnp.abs(f))

            @pl.when(gg + NB < NCH)
            def _(gg=gg, bi=bi):
                chunk_copy(gg + NB, bi).start()
        return tot

    bsum = jnp.sum(jnp.abs(b_v[...]))
    out_ref[...] = jnp.full((1, 1), tot / (1.0 + bsum), jnp.float32)


_call = pl.pallas_call(
    _body,
    in_specs=[
        pl.BlockSpec(memory_space=pl.ANY),
        pl.BlockSpec(memory_space=pl.ANY),
        pl.BlockSpec(memory_space=pl.ANY),
        pl.BlockSpec(memory_space=pl.ANY),
    ],
    out_specs=pl.BlockSpec(memory_space=pltpu.MemorySpace.VMEM),
    out_shape=jax.ShapeDtypeStruct((1, 1), jnp.float32),
    scratch_shapes=[
        pltpu.VMEM((NB, RB, N), jnp.float32),
        pltpu.VMEM((1, N), jnp.float32),
        pltpu.VMEM((1, M), jnp.float32),
        pltpu.VMEM((1, M), jnp.float32),
    ] + [pltpu.SemaphoreType.DMA] * (NB + 3),
)


def kernel(Q, A, AT, b, c, x, y, Iy):
    res = _call(A, x.reshape(1, N), b.reshape(1, M), Iy.reshape(1, M))
    return res[0, 0]
